# Initial kernel scaffold; baseline (speedup 1.0000x reference)
import jax
import jax.numpy as jnp
from jax.experimental import pallas as pl


def kernel(x, node_to_graph_map, W_score, b_score, W_val, b_val):
    raise NotImplementedError("write your pallas kernel here")


# speedup vs baseline: 3.6465x; 3.6465x over previous
import functools

import jax
import jax.numpy as jnp
from jax import lax
from jax.experimental import pallas as pl
from jax.experimental.pallas import tpu as pltpu
from jax.experimental.pallas import tpu_sc as plsc

NUM_G = 256
DIM = 128
HEADS = 4
HEAD_DIM = DIM // HEADS
BLOCK = 1024


def _pool_body(xb, mapb, ws, bs, wv, bv, rexp, repr_out, s_ref, u_ref):
    i = pl.program_id(0)
    nb = pl.num_programs(0)

    @pl.when(i == 0)
    def _init():
        s_ref[...] = jnp.zeros_like(s_ref)
        u_ref[...] = jnp.zeros_like(u_ref)

    x = xb[...]
    s = jnp.dot(x, ws[...], preferred_element_type=jnp.float32) + bs[...]
    v = jnp.dot(x, wv[...], preferred_element_type=jnp.float32) + bv[...]
    e = jnp.exp(s)
    idx = mapb[...][0]
    gids = lax.broadcasted_iota(jnp.int32, (NUM_G, 1), 0)
    ot = (gids == idx).astype(jnp.float32)
    s_ref[...] += jnp.dot(ot, e, preferred_element_type=jnp.float32)
    ef = jnp.dot(e, rexp[...], preferred_element_type=jnp.float32)
    u_ref[...] += jnp.dot(ot, ef * v, preferred_element_type=jnp.float32)

    @pl.when(i == nb - 1)
    def _finish():
        sfull = jnp.dot(s_ref[...], rexp[...], preferred_element_type=jnp.float32)
        repr_out[...] = u_ref[...] / (sfull + 1e-9)


def _pool(x_pad, map3, w_score, b_score, w_val, b_val, rexp):
    nb = x_pad.shape[0] // BLOCK
    return pl.pallas_call(
        _pool_body,
        grid=(nb,),
        in_specs=[
            pl.BlockSpec((BLOCK, DIM), lambda i: (i, 0)),
            pl.BlockSpec((1, 1, BLOCK), lambda i: (i, 0, 0)),
            pl.BlockSpec((DIM, HEADS), lambda i: (0, 0)),
            pl.BlockSpec((1, HEADS), lambda i: (0, 0)),
            pl.BlockSpec((DIM, DIM), lambda i: (0, 0)),
            pl.BlockSpec((1, DIM), lambda i: (0, 0)),
            pl.BlockSpec((HEADS, DIM), lambda i: (0, 0)),
        ],
        out_specs=pl.BlockSpec((NUM_G, DIM), lambda i: (0, 0)),
        out_shape=jax.ShapeDtypeStruct((NUM_G, DIM), jnp.float32),
        scratch_shapes=[
            pltpu.VMEM((NUM_G, HEADS), jnp.float32),
            pltpu.VMEM((NUM_G, DIM), jnp.float32),
        ],
        compiler_params=pltpu.CompilerParams(
            dimension_semantics=("arbitrary",),
        ),
    )(x_pad, map3, w_score, b_score, w_val, b_val, rexp)


def _lift(repr_, idx2d, n_pad):
    info = plsc.get_sparse_core_info()
    nc, ns = info.num_cores, info.num_subcores
    nw = nc * ns
    rows_per_w = n_pad // nw
    chunks = rows_per_w // 128

    @functools.partial(
        pl.kernel,
        mesh=plsc.VectorSubcoreMesh(core_axis_name="c", subcore_axis_name="s"),
        out_type=jax.ShapeDtypeStruct((n_pad, DIM), jnp.float32),
        scratch_types=[
            pltpu.VMEM((rows_per_w,), jnp.int32),
            pltpu.VMEM((128, DIM), jnp.float32),
            pltpu.SemaphoreType.DMA,
        ],
    )
    def lift_kernel(repr_hbm, idx_hbm, out_hbm, idx_v, rows_v, sem):
        wid = lax.axis_index("s") * nc + lax.axis_index("c")
        pltpu.sync_copy(idx_hbm.at[pl.ds(wid * rows_per_w, rows_per_w)], idx_v)
        for c in range(chunks):
            pltpu.async_copy(
                repr_hbm.at[idx_v.at[pl.ds(c * 128, 128)]], rows_v, sem
            ).wait()
            pltpu.sync_copy(rows_v, out_hbm.at[pl.ds(wid * rows_per_w + c * 128, 128)])

    return lift_kernel(repr_, idx2d)


def kernel(x, node_to_graph_map, W_score, b_score, W_val, b_val):
    n = x.shape[0]
    n_pad = ((n + 4095) // 4096) * 4096
    if n_pad % BLOCK:
        n_pad = ((n_pad + BLOCK - 1) // BLOCK) * BLOCK
    x_pad = jnp.pad(x, ((0, n_pad - n), (0, 0)))
    map_oh = jnp.pad(node_to_graph_map, (0, n_pad - n), constant_values=NUM_G)
    map3 = map_oh.reshape(n_pad // BLOCK, 1, BLOCK)
    map_g = jnp.pad(node_to_graph_map, (0, n_pad - n))
    rexp = (
        lax.broadcasted_iota(jnp.int32, (HEADS, DIM), 1) // HEAD_DIM
        == lax.broadcasted_iota(jnp.int32, (HEADS, DIM), 0)
    ).astype(jnp.float32)
    repr_ = _pool(
        x_pad, map3, W_score, b_score.reshape(1, HEADS), W_val, b_val.reshape(1, DIM), rexp
    )
    out = _lift(repr_, map_g, n_pad)
    return out[:n]


# speedup vs baseline: 3.8301x; 1.0504x over previous
import functools

import jax
import jax.numpy as jnp
from jax import lax
from jax.experimental import pallas as pl
from jax.experimental.pallas import tpu as pltpu
from jax.experimental.pallas import tpu_sc as plsc

NUM_G = 256
DIM = 128
HEADS = 4
HEAD_DIM = DIM // HEADS
BLOCK = 1024


def _pool_body(xb, mapb, ws, bs, wv, bv, rexp, repr_out, s_ref, u_ref):
    i = pl.program_id(0)
    nb = pl.num_programs(0)

    @pl.when(i == 0)
    def _init():
        s_ref[...] = jnp.zeros_like(s_ref)
        u_ref[...] = jnp.zeros_like(u_ref)

    x = xb[...]
    s = jnp.dot(x, ws[...], preferred_element_type=jnp.float32) + bs[...]
    v = jnp.dot(x, wv[...], preferred_element_type=jnp.float32) + bv[...]
    e = jnp.exp(s)
    idx = mapb[...][0]
    gids = lax.broadcasted_iota(jnp.int32, (NUM_G, 1), 0)
    ot = (gids == idx).astype(jnp.float32)
    s_ref[...] += jnp.dot(ot, e, preferred_element_type=jnp.float32)
    ef = jnp.dot(e, rexp[...], preferred_element_type=jnp.float32)
    u_ref[...] += jnp.dot(ot, ef * v, preferred_element_type=jnp.float32)

    @pl.when(i == nb - 1)
    def _finish():
        sfull = jnp.dot(s_ref[...], rexp[...], preferred_element_type=jnp.float32)
        repr_out[...] = u_ref[...] / (sfull + 1e-9)


def _pool(x_pad, map3, w_score, b_score, w_val, b_val, rexp):
    nb = x_pad.shape[0] // BLOCK
    return pl.pallas_call(
        _pool_body,
        grid=(nb,),
        in_specs=[
            pl.BlockSpec((BLOCK, DIM), lambda i: (i, 0)),
            pl.BlockSpec((1, 1, BLOCK), lambda i: (i, 0, 0)),
            pl.BlockSpec((DIM, HEADS), lambda i: (0, 0)),
            pl.BlockSpec((1, HEADS), lambda i: (0, 0)),
            pl.BlockSpec((DIM, DIM), lambda i: (0, 0)),
            pl.BlockSpec((1, DIM), lambda i: (0, 0)),
            pl.BlockSpec((HEADS, DIM), lambda i: (0, 0)),
        ],
        out_specs=pl.BlockSpec((NUM_G, DIM), lambda i: (0, 0)),
        out_shape=jax.ShapeDtypeStruct((NUM_G, DIM), jnp.float32),
        scratch_shapes=[
            pltpu.VMEM((NUM_G, HEADS), jnp.float32),
            pltpu.VMEM((NUM_G, DIM), jnp.float32),
        ],
        compiler_params=pltpu.CompilerParams(
            dimension_semantics=("arbitrary",),
        ),
    )(x_pad, map3, w_score, b_score, w_val, b_val, rexp)


GATHER_ROWS = 80
STEP_ROWS = 320


def _lift(repr_, idx2d, n_pad):
    info = plsc.get_sparse_core_info()
    nc, ns = info.num_cores, info.num_subcores
    nw = nc * ns
    rows_per_w = n_pad // nw
    steps = rows_per_w // STEP_ROWS
    gps = STEP_ROWS // GATHER_ROWS

    @functools.partial(
        pl.kernel,
        mesh=plsc.VectorSubcoreMesh(core_axis_name="c", subcore_axis_name="s"),
        out_type=jax.ShapeDtypeStruct((n_pad, DIM), jnp.float32),
        scratch_types=[
            pltpu.VMEM((rows_per_w,), jnp.int32),
            pltpu.VMEM((2, STEP_ROWS, DIM), jnp.float32),
            pltpu.SemaphoreType.DMA,
            pltpu.SemaphoreType.DMA,
        ],
    )
    def lift_kernel(repr_hbm, idx_hbm, out_hbm, idx_v, rows_v, gsem, ssem):
        wid = lax.axis_index("s") * nc + lax.axis_index("c")
        pltpu.sync_copy(idx_hbm.at[pl.ds(wid * rows_per_w, rows_per_w)], idx_v)

        def fire_gathers(p):
            buf = p % 2
            for g in range(gps):
                r = p * STEP_ROWS + g * GATHER_ROWS
                pltpu.async_copy(
                    repr_hbm.at[idx_v.at[pl.ds(r, GATHER_ROWS)]],
                    rows_v.at[buf].at[pl.ds(g * GATHER_ROWS, GATHER_ROWS)],
                    gsem,
                )

        fire_gathers(0)
        for p in range(steps):
            buf = p % 2
            for g in range(gps):
                pltpu.make_async_copy(
                    repr_hbm.at[idx_v.at[pl.ds(0, GATHER_ROWS)]],
                    rows_v.at[buf].at[pl.ds(g * GATHER_ROWS, GATHER_ROWS)],
                    gsem,
                ).wait()
            if p >= 1:
                pltpu.make_async_copy(
                    rows_v.at[(p - 1) % 2],
                    out_hbm.at[pl.ds(wid * rows_per_w + (p - 1) * STEP_ROWS, STEP_ROWS)],
                    ssem,
                ).wait()
            pltpu.async_copy(
                rows_v.at[buf],
                out_hbm.at[pl.ds(wid * rows_per_w + p * STEP_ROWS, STEP_ROWS)],
                ssem,
            )
            if p + 1 < steps:
                fire_gathers(p + 1)
        pltpu.make_async_copy(
            rows_v.at[(steps - 1) % 2],
            out_hbm.at[pl.ds(wid * rows_per_w + (steps - 1) * STEP_ROWS, STEP_ROWS)],
            ssem,
        ).wait()

    return lift_kernel(repr_, idx2d)


def kernel(x, node_to_graph_map, W_score, b_score, W_val, b_val):
    n = x.shape[0]
    n_pad = ((n + 4095) // 4096) * 4096
    if n_pad % BLOCK:
        n_pad = ((n_pad + BLOCK - 1) // BLOCK) * BLOCK
    x_pad = jnp.pad(x, ((0, n_pad - n), (0, 0)))
    map_oh = jnp.pad(node_to_graph_map, (0, n_pad - n), constant_values=NUM_G)
    map3 = map_oh.reshape(n_pad // BLOCK, 1, BLOCK)
    map_g = jnp.pad(node_to_graph_map, (0, n_pad - n))
    rexp = (
        lax.broadcasted_iota(jnp.int32, (HEADS, DIM), 1) // HEAD_DIM
        == lax.broadcasted_iota(jnp.int32, (HEADS, DIM), 0)
    ).astype(jnp.float32)
    repr_ = _pool(
        x_pad, map3, W_score, b_score.reshape(1, HEADS), W_val, b_val.reshape(1, DIM), rexp
    )
    out = _lift(repr_, map_g, n_pad)
    return out[:n]
